# baseline (speedup 1.0000x reference)
import jax
import jax.numpy as jnp
from jax import lax
from jax.experimental import pallas as pl
from jax.experimental.pallas import tpu as pltpu, tpu_sc as plsc

B = 4 * 8192
D = 512
NC = 2
NS = 16
NW = NC * NS
QPW = B // (4 * NW)
CHUNKQ = 16
NCHUNK = QPW // CHUNKQ
NBUF = 3
QD = 4 * D


def _sc_body(x_hbm, table_hbm, out_hbm, rep_hbm,
             idx_v, qidx_v, tbl_v, rows_v, gsem, osem):
    wid = lax.axis_index("s") * NC + lax.axis_index("c")
    pltpu.sync_copy(x_hbm.at[wid], idx_v)
    pltpu.sync_copy(table_hbm, tbl_v)

    reps = []
    for q in range(16):
        for p in range(4):
            bit = (q >> (3 - p)) & 1
            c = pltpu.make_async_copy(
                tbl_v.at[bit], rep_hbm.at[wid * 16 + q, pl.ds(p * D, D)], osem)
            c.start()
            reps.append(c)

    woff = wid * 16
    for c16 in range(NCHUNK):
        sl = pl.ds(c16 * CHUNKQ, CHUNKQ)
        qidx_v[sl] = (8 * idx_v[0, sl] + 4 * idx_v[1, sl]
                      + 2 * idx_v[2, sl] + idx_v[3, sl] + woff)

    for c in reps:
        c.wait()

    base = wid * QPW
    gathers = [None] * NCHUNK
    outs = [None] * NCHUNK
    for j in range(min(NBUF, NCHUNK)):
        gathers[j] = pltpu.make_async_copy(
            rep_hbm.at[qidx_v.at[pl.ds(j * CHUNKQ, CHUNKQ)]],
            rows_v.at[j % NBUF], gsem)
        gathers[j].start()
    for j in range(NCHUNK):
        b = j % NBUF
        gathers[j].wait()
        outs[j] = pltpu.make_async_copy(
            rows_v.at[b], out_hbm.at[pl.ds(base + j * CHUNKQ, CHUNKQ)], osem)
        outs[j].start()
        nj = j + NBUF
        if nj < NCHUNK:
            outs[j].wait()
            gathers[nj] = pltpu.make_async_copy(
                rep_hbm.at[qidx_v.at[pl.ds(nj * CHUNKQ, CHUNKQ)]],
                rows_v.at[b], gsem)
            gathers[nj].start()
    for j in range(max(0, NCHUNK - NBUF), NCHUNK):
        outs[j].wait()


def kernel(x, table):
    xq = x.reshape(NW, QPW, 4).transpose(0, 2, 1).astype(jnp.int32)
    out, _ = pl.kernel(
        _sc_body,
        out_type=[
            jax.ShapeDtypeStruct((B // 4, QD), jnp.float32),
            jax.ShapeDtypeStruct((NW * 16, QD), jnp.float32),
        ],
        mesh=plsc.VectorSubcoreMesh(core_axis_name="c", subcore_axis_name="s"),
        scratch_types=[
            pltpu.VMEM((4, QPW), jnp.int32),
            pltpu.VMEM((QPW,), jnp.int32),
            pltpu.VMEM((2, D), jnp.float32),
            pltpu.VMEM((NBUF, CHUNKQ, QD), jnp.float32),
            pltpu.SemaphoreType.DMA,
            pltpu.SemaphoreType.DMA,
        ],
    )(xq, table)
    return out.reshape(x.shape[0], x.shape[1], D)


# speedup vs baseline: 1.0053x; 1.0053x over previous
import jax
import jax.numpy as jnp
from jax import lax
from jax.experimental import pallas as pl
from jax.experimental.pallas import tpu as pltpu, tpu_sc as plsc

B = 4 * 8192
D = 512
NC = 2
NS = 16
NW = NC * NS
QPW = B // (4 * NW)
CHUNKQ = 16
NCHUNK = QPW // CHUNKQ
NBUF = 3
QD = 4 * D


def _sc_body(x_hbm, table_hbm, out_hbm, rep_hbm,
             idx_v, qidx_v, tbl_v, rows_v, gsem, osem):
    wid = lax.axis_index("s") * NC + lax.axis_index("c")
    pltpu.sync_copy(x_hbm.at[wid], idx_v)
    pltpu.sync_copy(table_hbm, tbl_v)

    reps = []
    for q in range(16):
        for p in range(4):
            bit = (q >> (3 - p)) & 1
            c = pltpu.make_async_copy(
                tbl_v.at[bit], rep_hbm.at[wid * 16 + q, pl.ds(p * D, D)], osem)
            c.start()
            reps.append(c)

    woff = wid * 16
    for c16 in range(NCHUNK):
        sl = pl.ds(c16 * CHUNKQ, CHUNKQ)
        qidx_v[c16, :] = (8 * idx_v[0, sl] + 4 * idx_v[1, sl]
                          + 2 * idx_v[2, sl] + idx_v[3, sl] + woff)

    for c in reps:
        c.wait()

    base = wid * QPW
    gathers = [None] * NCHUNK
    outs = [None] * NCHUNK
    for j in range(min(NBUF, NCHUNK)):
        gathers[j] = pltpu.make_async_copy(
            rep_hbm.at[qidx_v.at[j]],
            rows_v.at[j % NBUF], gsem)
        gathers[j].start()
    for j in range(NCHUNK):
        b = j % NBUF
        gathers[j].wait()
        outs[j] = pltpu.make_async_copy(
            rows_v.at[b], out_hbm.at[pl.ds(base + j * CHUNKQ, CHUNKQ)], osem)
        outs[j].start()
        nj = j + NBUF
        if nj < NCHUNK:
            outs[j].wait()
            gathers[nj] = pltpu.make_async_copy(
                rep_hbm.at[qidx_v.at[nj]],
                rows_v.at[b], gsem)
            gathers[nj].start()
    for j in range(max(0, NCHUNK - NBUF), NCHUNK):
        outs[j].wait()


def kernel(x, table):
    xq = x.reshape(NW, QPW, 4).transpose(0, 2, 1).astype(jnp.int32)
    out, _ = pl.kernel(
        _sc_body,
        out_type=[
            jax.ShapeDtypeStruct((B // 4, QD), jnp.float32),
            jax.ShapeDtypeStruct((NW * 16, QD), jnp.float32),
        ],
        mesh=plsc.VectorSubcoreMesh(core_axis_name="c", subcore_axis_name="s"),
        scratch_types=[
            pltpu.VMEM((4, QPW), jnp.int32),
            pltpu.VMEM((NCHUNK, CHUNKQ), jnp.int32),
            pltpu.VMEM((2, D), jnp.float32),
            pltpu.VMEM((NBUF, CHUNKQ, QD), jnp.float32),
            pltpu.SemaphoreType.DMA,
            pltpu.SemaphoreType.DMA,
        ],
    )(xq, table)
    return out.reshape(x.shape[0], x.shape[1], D)
